# baseline (speedup 1.0000x reference)
import jax
import jax.numpy as jnp
from jax import lax
from jax.experimental import pallas as pl
from jax.experimental.pallas import tpu as pltpu

_EPS = 1e-5


def _conv1_stats_kernel(x_ref, t1_ref, z_ref, sum_ref, ssq_ref, xscr_ref):
    NB, C0, H, W = x_ref.shape
    H1 = H // 2
    CW1 = t1_ref.shape[-1]

    xscr_ref[:, :, :, 0:8, :] = jnp.zeros((NB, C0, 2, 8, W), jnp.float32)
    xscr_ref[:, :, 0, 8:8 + H1, :] = x_ref[:, :, pl.ds(0, H1, 2), :]
    xscr_ref[:, :, 1, 8:8 + H1, :] = x_ref[:, :, pl.ds(1, H1, 2), :]

    CH = NB
    s = jnp.zeros((1, CW1), jnp.float32)
    q = jnp.zeros((1, CW1), jnp.float32)
    for c in range(0, NB, CH):
        taps = []
        for kh in range(3):
            par, base = ((1, 7), (0, 8), (1, 8))[kh]
            for cin in range(C0):
                t = xscr_ref[c:c + CH, cin, par, pl.ds(base, H1), :]
                taps.append(t.astype(jnp.bfloat16).reshape(CH * H1, W))
        tap = jnp.concatenate(taps, axis=1)
        acc = jnp.dot(tap, t1_ref[...],
                      preferred_element_type=jnp.float32)
        z_ref[c:c + CH] = acc.reshape(CH, H1, CW1).astype(z_ref.dtype)
        s = s + jnp.sum(acc, axis=0, keepdims=True)
        q = q + jnp.sum(acc * acc, axis=0, keepdims=True)
    sum_ref[...] = s
    ssq_ref[...] = q


def _bn_relu_conv2_kernel(z_ref, sc_ref, sh_ref, t2_ref, y2_ref, y1_ref,
                          yscr_ref):
    NB, H1, CW1 = z_ref.shape
    H2 = H1 // 2
    CW2 = t2_ref.shape[-1]
    C1 = y1_ref.shape[1]
    W1 = CW1 // C1
    C2 = y2_ref.shape[1]
    W2 = CW2 // C2

    LG = CW1 // 128
    CH = min(16, NB)
    for c in range(0, NB, CH):
        yc = jnp.maximum(z_ref[c:c + CH] * sc_ref[...] + sh_ref[...], 0.0)
        y1_ref[c:c + CH] = jnp.transpose(
            yc.reshape(CH, H1, C1, W1), (0, 2, 1, 3)).reshape(CH, C1,
                                                              H1 * W1)
        yscr_ref[c:c + CH, 0:8, :, :] = jnp.zeros((CH, 8, LG, 128),
                                                  jnp.float32)
        yscr_ref[c:c + CH, 8:H1 + 8, :, :] = yc.reshape(CH, H1, LG, 128)

        taps = [
            yscr_ref[c:c + CH, pl.ds(7 + kh, H2, 2), :, :].astype(
                jnp.bfloat16).reshape(CH * H2, CW1)
            for kh in range(3)
        ]
        tap = jnp.concatenate(taps, axis=1)
        acc = jnp.dot(tap, t2_ref[...],
                      preferred_element_type=jnp.float32)
        y2_ref[c:c + CH] = jnp.transpose(
            acc.reshape(CH, H2, C2, W2), (0, 2, 1, 3)).reshape(CH, C2,
                                                               H2 * W2)


def _band_matrices(w_hwio, w_in, w_out):
    kw = jnp.arange(3)[:, None, None]
    w_ = jnp.arange(w_in)[None, :, None]
    j_ = jnp.arange(w_out)[None, None, :]
    sel = (w_ == 2 * j_ + kw - 1).astype(jnp.float32)
    t = jnp.einsum("qwj,hqcf->hcwfj", sel, w_hwio.astype(jnp.float32))
    cin, cout = w_hwio.shape[2], w_hwio.shape[3]
    return t.reshape(3, cin, w_in, cout * w_out).astype(jnp.bfloat16)


def kernel(x_nchw, w1, gamma, beta, w2):
    N, C0, H, W = x_nchw.shape
    C1 = w1.shape[-1]
    C2 = w2.shape[-1]
    H1, W1 = H // 2, W // 2
    H2, W2 = H1 // 2, W1 // 2
    CW1, CW2 = C1 * W1, C2 * W2

    NB = 16
    while N % NB:
        NB -= 1
    G = N // NB

    x = x_nchw.astype(jnp.float32)
    t1 = _band_matrices(w1, W, W1).reshape(9 * W, CW1)
    t2 = _band_matrices(w2, W1, W2).reshape(3 * CW1, CW2)

    cparams = pltpu.CompilerParams(
        dimension_semantics=("parallel",),
        vmem_limit_bytes=64 * 1024 * 1024,
    )

    z, psum, pssq = pl.pallas_call(
        _conv1_stats_kernel,
        out_shape=(
            jax.ShapeDtypeStruct((N, H1, CW1), jnp.bfloat16),
            jax.ShapeDtypeStruct((G, 1, CW1), jnp.float32),
            jax.ShapeDtypeStruct((G, 1, CW1), jnp.float32),
        ),
        grid_spec=pltpu.PrefetchScalarGridSpec(
            num_scalar_prefetch=0,
            grid=(G,),
            in_specs=[
                pl.BlockSpec((NB, C0, H, W), lambda n: (n, 0, 0, 0)),
                pl.BlockSpec((9 * W, CW1), lambda n: (0, 0)),
            ],
            out_specs=(
                pl.BlockSpec((NB, H1, CW1), lambda n: (n, 0, 0)),
                pl.BlockSpec((pl.Squeezed(), 1, CW1), lambda n: (n, 0, 0)),
                pl.BlockSpec((pl.Squeezed(), 1, CW1), lambda n: (n, 0, 0)),
            ),
            scratch_shapes=[pltpu.VMEM((NB, C0, 2, H // 2 + 8, W),
                                       jnp.float32)],
        ),
        compiler_params=cparams,
    )(x, t1)

    cnt = jnp.float32(N * H1 * W1)
    mean = jnp.sum(psum[:, 0, :], axis=0).reshape(C1, W1).sum(axis=1) / cnt
    ex2 = jnp.sum(pssq[:, 0, :], axis=0).reshape(C1, W1).sum(axis=1) / cnt
    var = jnp.maximum(ex2 - mean * mean, 0.0)
    scale = gamma.astype(jnp.float32) * lax.rsqrt(var + _EPS)
    shift = beta.astype(jnp.float32) - mean * scale
    sc_e = jnp.repeat(scale, W1).reshape(1, CW1)
    sh_e = jnp.repeat(shift, W1).reshape(1, CW1)

    y2, y1 = pl.pallas_call(
        _bn_relu_conv2_kernel,
        out_shape=(
            jax.ShapeDtypeStruct((N, C2, H2 * W2), jnp.float32),
            jax.ShapeDtypeStruct((N, C1, H1 * W1), jnp.float32),
        ),
        grid_spec=pltpu.PrefetchScalarGridSpec(
            num_scalar_prefetch=0,
            grid=(G,),
            in_specs=[
                pl.BlockSpec((NB, H1, CW1), lambda n: (n, 0, 0)),
                pl.BlockSpec((1, CW1), lambda n: (0, 0)),
                pl.BlockSpec((1, CW1), lambda n: (0, 0)),
                pl.BlockSpec((3 * CW1, CW2), lambda n: (0, 0)),
            ],
            out_specs=(
                pl.BlockSpec((NB, C2, H2 * W2), lambda n: (n, 0, 0)),
                pl.BlockSpec((NB, C1, H1 * W1), lambda n: (n, 0, 0)),
            ),
            scratch_shapes=[pltpu.VMEM((NB, H1 + 8, CW1 // 128, 128),
                                       jnp.float32)],
        ),
        compiler_params=cparams,
    )(z, sc_e, sh_e, t2)

    return y2.reshape(N, C2, H2, W2), y1.reshape(N, C1, H1, W1)


# speedup vs baseline: 1.0616x; 1.0616x over previous
import jax
import jax.numpy as jnp
from jax import lax
from jax.experimental import pallas as pl
from jax.experimental.pallas import tpu as pltpu

_EPS = 1e-5


def _conv1_stats_kernel(x_ref, t1_ref, z_ref, sum_ref, ssq_ref, xscr_ref):
    NB, C0, H, W = x_ref.shape
    H1 = H // 2
    CW1 = t1_ref.shape[-1]

    xscr_ref[:, :, :, 0:8, :] = jnp.zeros((NB, C0, 2, 8, W), jnp.float32)
    xscr_ref[:, :, 0, 8:8 + H1, :] = x_ref[:, :, pl.ds(0, H1, 2), :]
    xscr_ref[:, :, 1, 8:8 + H1, :] = x_ref[:, :, pl.ds(1, H1, 2), :]

    CH = NB
    s = jnp.zeros((1, CW1), jnp.float32)
    q = jnp.zeros((1, CW1), jnp.float32)
    for c in range(0, NB, CH):
        taps = []
        for kh in range(3):
            par, base = ((1, 7), (0, 8), (1, 8))[kh]
            for cin in range(C0):
                t = xscr_ref[c:c + CH, cin, par, pl.ds(base, H1), :]
                taps.append(t.astype(jnp.bfloat16).reshape(CH * H1, W))
        tap = jnp.concatenate(taps, axis=1)
        acc = jnp.dot(tap, t1_ref[...],
                      preferred_element_type=jnp.float32)
        z_ref[c:c + CH] = acc.reshape(CH, H1, CW1).astype(z_ref.dtype)
        s = s + jnp.sum(acc, axis=0, keepdims=True)
        q = q + jnp.sum(acc * acc, axis=0, keepdims=True)
    sum_ref[...] = s
    ssq_ref[...] = q


def _bn_relu_conv2_kernel(z_ref, sc_ref, sh_ref, t2_ref, y2_ref, y1_ref,
                          yscr_ref):
    NB, H1, CW1 = z_ref.shape
    H2 = H1 // 2
    CW2 = t2_ref.shape[-1]
    C1 = y1_ref.shape[1]
    W1 = CW1 // C1
    C2 = y2_ref.shape[1]
    W2 = CW2 // C2

    LG = CW1 // 128
    CH = min(16, NB)
    for c in range(0, NB, CH):
        yc = jnp.maximum(z_ref[c:c + CH] * sc_ref[...] + sh_ref[...], 0.0)
        yb1 = yc.astype(jnp.bfloat16)
        y1_ref[c:c + CH] = jnp.transpose(
            yb1.reshape(CH, H1, C1, W1), (0, 2, 1, 3)).reshape(
                CH, C1, H1 * W1).astype(jnp.float32)
        yscr_ref[c:c + CH, 0:8, :, :] = jnp.zeros((CH, 8, LG, 128),
                                                  jnp.float32)
        yscr_ref[c:c + CH, 8:H1 + 8, :, :] = yc.reshape(CH, H1, LG, 128)

        taps = [
            yscr_ref[c:c + CH, pl.ds(7 + kh, H2, 2), :, :].astype(
                jnp.bfloat16).reshape(CH * H2, CW1)
            for kh in range(3)
        ]
        tap = jnp.concatenate(taps, axis=1)
        acc = jnp.dot(tap, t2_ref[...],
                      preferred_element_type=jnp.float32)
        ab = acc.astype(jnp.bfloat16)
        y2_ref[c:c + CH] = jnp.transpose(
            ab.reshape(CH, H2, C2, W2), (0, 2, 1, 3)).reshape(
                CH, C2, H2 * W2).astype(jnp.float32)


def _band_matrices(w_hwio, w_in, w_out):
    kw = jnp.arange(3)[:, None, None]
    w_ = jnp.arange(w_in)[None, :, None]
    j_ = jnp.arange(w_out)[None, None, :]
    sel = (w_ == 2 * j_ + kw - 1).astype(jnp.float32)
    t = jnp.einsum("qwj,hqcf->hcwfj", sel, w_hwio.astype(jnp.float32))
    cin, cout = w_hwio.shape[2], w_hwio.shape[3]
    return t.reshape(3, cin, w_in, cout * w_out).astype(jnp.bfloat16)


def kernel(x_nchw, w1, gamma, beta, w2):
    N, C0, H, W = x_nchw.shape
    C1 = w1.shape[-1]
    C2 = w2.shape[-1]
    H1, W1 = H // 2, W // 2
    H2, W2 = H1 // 2, W1 // 2
    CW1, CW2 = C1 * W1, C2 * W2

    NB = 16
    while N % NB:
        NB -= 1
    G = N // NB

    x = x_nchw.astype(jnp.float32)
    t1 = _band_matrices(w1, W, W1).reshape(9 * W, CW1)
    t2 = _band_matrices(w2, W1, W2).reshape(3 * CW1, CW2)

    cparams = pltpu.CompilerParams(
        dimension_semantics=("parallel",),
        vmem_limit_bytes=64 * 1024 * 1024,
    )

    z, psum, pssq = pl.pallas_call(
        _conv1_stats_kernel,
        out_shape=(
            jax.ShapeDtypeStruct((N, H1, CW1), jnp.bfloat16),
            jax.ShapeDtypeStruct((G, 1, CW1), jnp.float32),
            jax.ShapeDtypeStruct((G, 1, CW1), jnp.float32),
        ),
        grid_spec=pltpu.PrefetchScalarGridSpec(
            num_scalar_prefetch=0,
            grid=(G,),
            in_specs=[
                pl.BlockSpec((NB, C0, H, W), lambda n: (n, 0, 0, 0)),
                pl.BlockSpec((9 * W, CW1), lambda n: (0, 0)),
            ],
            out_specs=(
                pl.BlockSpec((NB, H1, CW1), lambda n: (n, 0, 0)),
                pl.BlockSpec((pl.Squeezed(), 1, CW1), lambda n: (n, 0, 0)),
                pl.BlockSpec((pl.Squeezed(), 1, CW1), lambda n: (n, 0, 0)),
            ),
            scratch_shapes=[pltpu.VMEM((NB, C0, 2, H // 2 + 8, W),
                                       jnp.float32)],
        ),
        compiler_params=cparams,
    )(x, t1)

    cnt = jnp.float32(N * H1 * W1)
    mean = jnp.sum(psum[:, 0, :], axis=0).reshape(C1, W1).sum(axis=1) / cnt
    ex2 = jnp.sum(pssq[:, 0, :], axis=0).reshape(C1, W1).sum(axis=1) / cnt
    var = jnp.maximum(ex2 - mean * mean, 0.0)
    scale = gamma.astype(jnp.float32) * lax.rsqrt(var + _EPS)
    shift = beta.astype(jnp.float32) - mean * scale
    sc_e = jnp.repeat(scale, W1).reshape(1, CW1)
    sh_e = jnp.repeat(shift, W1).reshape(1, CW1)

    y2, y1 = pl.pallas_call(
        _bn_relu_conv2_kernel,
        out_shape=(
            jax.ShapeDtypeStruct((N, C2, H2 * W2), jnp.float32),
            jax.ShapeDtypeStruct((N, C1, H1 * W1), jnp.float32),
        ),
        grid_spec=pltpu.PrefetchScalarGridSpec(
            num_scalar_prefetch=0,
            grid=(G,),
            in_specs=[
                pl.BlockSpec((NB, H1, CW1), lambda n: (n, 0, 0)),
                pl.BlockSpec((1, CW1), lambda n: (0, 0)),
                pl.BlockSpec((1, CW1), lambda n: (0, 0)),
                pl.BlockSpec((3 * CW1, CW2), lambda n: (0, 0)),
            ],
            out_specs=(
                pl.BlockSpec((NB, C2, H2 * W2), lambda n: (n, 0, 0)),
                pl.BlockSpec((NB, C1, H1 * W1), lambda n: (n, 0, 0)),
            ),
            scratch_shapes=[pltpu.VMEM((NB, H1 + 8, CW1 // 128, 128),
                                       jnp.float32)],
        ),
        compiler_params=cparams,
    )(z, sc_e, sh_e, t2)

    return y2.reshape(N, C2, H2, W2), y1.reshape(N, C1, H1, W1)
